# baseline (speedup 1.0000x reference)
import jax
import jax.numpy as jnp
from jax.experimental import pallas as pl
from jax.experimental.pallas import tpu as pltpu

N_NODES = 52
EPS = 1e-5
C2 = 128
BT = 512
INV = 1.0 / (1.0 + EPS) ** 0.5


def _fused_kernel(x_ref, wg_ref, wn_ref, bg_ref, bn_ref, wc_ref, bc_ref,
                  g1_ref, be1_ref, w1_ref, b1_ref, go_ref, bo_ref,
                  w2_ref, b2_ref, out_ref,
                  w01_s, c0_s, wc_s, c1_s, w1_s, b1_s):
    bt = x_ref.shape[0]

    @pl.when(pl.program_id(0) == 0)
    def _fold_weights():
        w01_s[...] = jnp.dot(wg_ref[...], wn_ref[...],
                             preferred_element_type=jnp.float32)
        c0_s[...] = (jnp.dot(bg_ref[...], wn_ref[...],
                             preferred_element_type=jnp.float32)
                     + bn_ref[...]).astype(jnp.bfloat16)
        wc_s[...] = (g1_ref[...] * INV * wc_ref[...]).astype(jnp.bfloat16)
        c1_s[...] = (bc_ref[...] * g1_ref[...] * INV
                     + be1_ref[...]).astype(jnp.bfloat16)
        w1_s[...] = (w1_ref[...] * (go_ref[...] * INV)).astype(jnp.bfloat16)
        b1_s[...] = b1_ref[...] * go_ref[...] * INV + bo_ref[...]

    h3 = jax.lax.dot_general(x_ref[...], w01_s[...], (((1,), (0,)), ((), ())),
                             preferred_element_type=jnp.float32)
    h3b = jnp.maximum(h3.astype(jnp.bfloat16) + c0_s[...][None],
                      jnp.bfloat16(0))
    wcb = jnp.broadcast_to(wc_s[...], (bt, N_NODES, N_NODES))
    y = jax.lax.dot_general(wcb, h3b, (((2,), (1,)), ((0,), (0,))),
                            preferred_element_type=jnp.float32)
    yb = jnp.maximum(y.astype(jnp.bfloat16) + c1_s[...][None],
                     jnp.bfloat16(0))
    yf = yb.reshape(bt, N_NODES * C2)
    z = jnp.dot(yf, w1_s[...], preferred_element_type=jnp.float32)
    z = jnp.maximum(z + b1_s[...], 0.0)
    out_ref[...] = jnp.dot(z, w2_ref[...],
                           preferred_element_type=jnp.float32) + b2_ref[...]


@jax.jit
def kernel(x, W_gat0, att_src0, att_dst0, b_gat0, W_gcn1, b_gcn1, Wc, bc,
           g1, be1, W1, b1, g_o, b_o, W2, b2):
    B = x.shape[0]
    num_tiles = B // BT

    grid_spec = pltpu.PrefetchScalarGridSpec(
        num_scalar_prefetch=0,
        grid=(num_tiles,),
        in_specs=[
            pl.BlockSpec((BT, 14, N_NODES), lambda i: (i, 0, 0)),
            pl.BlockSpec((14, 64), lambda i: (0, 0)),
            pl.BlockSpec((64, C2), lambda i: (0, 0)),
            pl.BlockSpec((1, 64), lambda i: (0, 0)),
            pl.BlockSpec((1, C2), lambda i: (0, 0)),
            pl.BlockSpec((N_NODES, N_NODES), lambda i: (0, 0)),
            pl.BlockSpec((N_NODES, 1), lambda i: (0, 0)),
            pl.BlockSpec((N_NODES, 1), lambda i: (0, 0)),
            pl.BlockSpec((N_NODES, 1), lambda i: (0, 0)),
            pl.BlockSpec((N_NODES * C2, 64), lambda i: (0, 0)),
            pl.BlockSpec((1, 64), lambda i: (0, 0)),
            pl.BlockSpec((1, 64), lambda i: (0, 0)),
            pl.BlockSpec((1, 64), lambda i: (0, 0)),
            pl.BlockSpec((64, 10), lambda i: (0, 0)),
            pl.BlockSpec((1, 10), lambda i: (0, 0)),
        ],
        out_specs=pl.BlockSpec((BT, 10), lambda i: (i, 0)),
        scratch_shapes=[
            pltpu.VMEM((14, C2), jnp.float32),
            pltpu.VMEM((1, C2), jnp.bfloat16),
            pltpu.VMEM((N_NODES, N_NODES), jnp.bfloat16),
            pltpu.VMEM((N_NODES, 1), jnp.bfloat16),
            pltpu.VMEM((N_NODES * C2, 64), jnp.bfloat16),
            pltpu.VMEM((1, 64), jnp.float32),
        ],
    )
    out = pl.pallas_call(
        _fused_kernel,
        grid_spec=grid_spec,
        out_shape=jax.ShapeDtypeStruct((B, 10), jnp.float32),
        compiler_params=pltpu.CompilerParams(
            dimension_semantics=("arbitrary",),
        ),
    )(x, W_gat0, W_gcn1, b_gat0.reshape(1, 64), b_gcn1.reshape(1, C2),
      Wc, bc.reshape(N_NODES, 1), g1.reshape(N_NODES, 1),
      be1.reshape(N_NODES, 1), W1,
      b1.reshape(1, 64), g_o.reshape(1, 64), b_o.reshape(1, 64),
      W2, b2.reshape(1, 10))
    return out


# speedup vs baseline: 1.1018x; 1.1018x over previous
import jax
import jax.numpy as jnp
from jax.experimental import pallas as pl
from jax.experimental.pallas import tpu as pltpu

N_NODES = 52
EPS = 1e-5
C2 = 128
BT = 256
INV = 1.0 / (1.0 + EPS) ** 0.5


def _fused_kernel(x_ref, wg_ref, wn_ref, bg_ref, bn_ref, wc_ref, bc_ref,
                  g1_ref, be1_ref, w1_ref, b1_ref, go_ref, bo_ref,
                  w2_ref, b2_ref, out_ref,
                  w01_s, c0_s, wc_s, c1_s, w1_s, b1_s):
    bt = x_ref.shape[0]

    @pl.when(pl.program_id(0) == 0)
    def _fold_weights():
        w01_s[...] = jnp.dot(wg_ref[...], wn_ref[...],
                             preferred_element_type=jnp.float32)
        c0_s[...] = (jnp.dot(bg_ref[...], wn_ref[...],
                             preferred_element_type=jnp.float32)
                     + bn_ref[...]).astype(jnp.bfloat16)
        wc_s[...] = (g1_ref[...] * INV * wc_ref[...]).astype(jnp.bfloat16)
        c1_s[...] = (bc_ref[...] * g1_ref[...] * INV
                     + be1_ref[...]).astype(jnp.bfloat16)
        w1_s[...] = (w1_ref[...] * (go_ref[...] * INV)).astype(jnp.bfloat16)
        b1_s[...] = b1_ref[...] * go_ref[...] * INV + bo_ref[...]

    h3 = jax.lax.dot_general(x_ref[...], w01_s[...], (((1,), (0,)), ((), ())),
                             preferred_element_type=jnp.float32)
    h3b = jnp.maximum(h3.astype(jnp.bfloat16) + c0_s[...][None],
                      jnp.bfloat16(0))
    wcb = jnp.broadcast_to(wc_s[...], (bt, N_NODES, N_NODES))
    y = jax.lax.dot_general(wcb, h3b, (((2,), (1,)), ((0,), (0,))),
                            preferred_element_type=jnp.float32)
    yb = jnp.maximum(y.astype(jnp.bfloat16) + c1_s[...][None],
                     jnp.bfloat16(0))
    yf = yb.reshape(bt, N_NODES * C2)
    z = jnp.dot(yf, w1_s[...], preferred_element_type=jnp.float32)
    z = jnp.maximum(z + b1_s[...], 0.0)
    out_ref[...] = jnp.dot(z, w2_ref[...],
                           preferred_element_type=jnp.float32) + b2_ref[...]


@jax.jit
def kernel(x, W_gat0, att_src0, att_dst0, b_gat0, W_gcn1, b_gcn1, Wc, bc,
           g1, be1, W1, b1, g_o, b_o, W2, b2):
    B = x.shape[0]
    num_tiles = B // BT

    grid_spec = pltpu.PrefetchScalarGridSpec(
        num_scalar_prefetch=0,
        grid=(num_tiles,),
        in_specs=[
            pl.BlockSpec((BT, 14, N_NODES), lambda i: (i, 0, 0)),
            pl.BlockSpec((14, 64), lambda i: (0, 0)),
            pl.BlockSpec((64, C2), lambda i: (0, 0)),
            pl.BlockSpec((1, 64), lambda i: (0, 0)),
            pl.BlockSpec((1, C2), lambda i: (0, 0)),
            pl.BlockSpec((N_NODES, N_NODES), lambda i: (0, 0)),
            pl.BlockSpec((N_NODES, 1), lambda i: (0, 0)),
            pl.BlockSpec((N_NODES, 1), lambda i: (0, 0)),
            pl.BlockSpec((N_NODES, 1), lambda i: (0, 0)),
            pl.BlockSpec((N_NODES * C2, 64), lambda i: (0, 0)),
            pl.BlockSpec((1, 64), lambda i: (0, 0)),
            pl.BlockSpec((1, 64), lambda i: (0, 0)),
            pl.BlockSpec((1, 64), lambda i: (0, 0)),
            pl.BlockSpec((64, 10), lambda i: (0, 0)),
            pl.BlockSpec((1, 10), lambda i: (0, 0)),
        ],
        out_specs=pl.BlockSpec((BT, 10), lambda i: (i, 0)),
        scratch_shapes=[
            pltpu.VMEM((14, C2), jnp.float32),
            pltpu.VMEM((1, C2), jnp.bfloat16),
            pltpu.VMEM((N_NODES, N_NODES), jnp.bfloat16),
            pltpu.VMEM((N_NODES, 1), jnp.bfloat16),
            pltpu.VMEM((N_NODES * C2, 64), jnp.bfloat16),
            pltpu.VMEM((1, 64), jnp.float32),
        ],
    )
    out = pl.pallas_call(
        _fused_kernel,
        grid_spec=grid_spec,
        out_shape=jax.ShapeDtypeStruct((B, 10), jnp.float32),
        compiler_params=pltpu.CompilerParams(
            dimension_semantics=("arbitrary",),
        ),
    )(x, W_gat0, W_gcn1, b_gat0.reshape(1, 64), b_gcn1.reshape(1, C2),
      Wc, bc.reshape(N_NODES, 1), g1.reshape(N_NODES, 1),
      be1.reshape(N_NODES, 1), W1,
      b1.reshape(1, 64), g_o.reshape(1, 64), b_o.reshape(1, 64),
      W2, b2.reshape(1, 10))
    return out
